# baseline (speedup 1.0000x reference)
import functools

import jax
import jax.numpy as jnp
from jax import lax
from jax.experimental import pallas as pl
from jax.experimental.pallas import tpu as pltpu
from jax.experimental.pallas import tpu_sc as plsc

NC = 2
NS = 16
NW = NC * NS
LANES = 16
K = 80
CB = 8
BR = 400


def _mesh():
    return plsc.VectorSubcoreMesh(core_axis_name="c", subcore_axis_name="s")


@functools.lru_cache(maxsize=None)
def _build_hist(E_PAD, N_PAD):
    EPW = E_PAD // NW
    NCH = EPW // K
    NB = NCH // CB
    ZH = N_PAD // NS
    ZF = ((ZH + LANES - 1) // LANES) * LANES
    FIRE = 4
    f32 = jnp.float32
    sds = jax.ShapeDtypeStruct

    @functools.partial(
        pl.kernel,
        out_type=(sds((N_PAD,), f32),) * 4,
        mesh=_mesh(),
        scratch_types=[
            pltpu.VMEM((NB, CB, K), jnp.int32),
            pltpu.VMEM((NB, CB, K), jnp.int32),
            pltpu.VMEM((K,), f32),
            pltpu.VMEM((ZF,), f32),
            pltpu.VMEM_SHARED((N_PAD,), f32),
            pltpu.VMEM_SHARED((N_PAD,), f32),
            pltpu.SemaphoreType.DMA,
        ],
    )
    def hist_kernel(src_hbm, dst_hbm, hs0, hd0, hs1, hd1,
                    sidx, didx, ones_v, zb, hist_s, hist_d, sem):
        c = lax.axis_index("c")
        s = lax.axis_index("s")
        wid = s * NC + c

        def fill_z(i, _):
            zb[pl.ds(i * LANES, LANES)] = jnp.zeros((LANES,), f32)
            return 0
        lax.fori_loop(0, ZF // LANES, fill_z, 0)

        def fill_o(i, _):
            ones_v[pl.ds(i * LANES, LANES)] = jnp.ones((LANES,), f32)
            return 0
        lax.fori_loop(0, K // LANES, fill_o, 0)

        r0 = pl.multiple_of(s * ZH, 8)
        pltpu.sync_copy(zb.at[pl.ds(0, ZH)], hist_s.at[pl.ds(r0, ZH)])
        pltpu.sync_copy(zb.at[pl.ds(0, ZH)], hist_d.at[pl.ds(r0, ZH)])

        pltpu.sync_copy(src_hbm.at[wid], sidx)
        pltpu.sync_copy(dst_hbm.at[wid], didx)
        plsc.subcore_barrier()

        def fire_block(ob, _):
            for j in range(CB):
                pltpu.async_copy(
                    ones_v, hist_s.at[sidx.at[ob, j]], sem, add=True)
                pltpu.async_copy(
                    ones_v, hist_d.at[didx.at[ob, j]], sem, add=True)
                if j % FIRE == FIRE - 1:
                    for _k in range(2 * FIRE):
                        pltpu.make_async_copy(
                            ones_v, hist_s.at[sidx.at[0, 0]], sem).wait()
            return 0
        lax.fori_loop(0, NB, fire_block, 0)
        plsc.subcore_barrier()

        @pl.when(c == 0)
        def _():
            pltpu.sync_copy(hist_s.at[pl.ds(r0, ZH)], hs0.at[pl.ds(r0, ZH)])
            pltpu.sync_copy(hist_d.at[pl.ds(r0, ZH)], hd0.at[pl.ds(r0, ZH)])

        @pl.when(c == 1)
        def _():
            pltpu.sync_copy(hist_s.at[pl.ds(r0, ZH)], hs1.at[pl.ds(r0, ZH)])
            pltpu.sync_copy(hist_d.at[pl.ds(r0, ZH)], hd1.at[pl.ds(r0, ZH)])

    return hist_kernel


@functools.lru_cache(maxsize=None)
def _build_agg(E_PAD, N, N_PAD, D):
    EPW = E_PAD // NW
    NCH = EPW // K
    NB = NCH // CB
    RPT = N_PAD // NS
    ZCH = []
    left = RPT
    while left > 0:
        step = min(160, left)
        ZCH.append(step)
        left -= step
    f32 = jnp.float32
    sds = jax.ShapeDtypeStruct

    @functools.partial(
        pl.kernel,
        out_type=(sds((N_PAD, D), f32), sds((N_PAD, D), f32)),
        mesh=_mesh(),
        scratch_types=[
            pltpu.VMEM((2, CB, K), jnp.int32),
            pltpu.VMEM((2, CB, K), jnp.int32),
            pltpu.VMEM((2 * K, D), f32),
            pltpu.VMEM_SHARED((N_PAD, D), f32),
            pltpu.SemaphoreType.DMA,
            pltpu.SemaphoreType.DMA,
            pltpu.SemaphoreType.DMA,
            pltpu.SemaphoreType.DMA,
        ],
    )
    def agg_kernel(h_hbm, src_hbm, dst_hbm, agg0, agg1,
                   sidx, didx, rows, acc, semg0, semg1, sems, semi):
        c = lax.axis_index("c")
        s = lax.axis_index("s")
        wid = s * NC + c

        def fill_z(r, _):
            for jj in range(D // LANES):
                rows[r, pl.ds(jj * LANES, LANES)] = jnp.zeros((LANES,), f32)
            return 0
        lax.fori_loop(0, max(ZCH), fill_z, 0)

        base_r = s * RPT
        off = 0
        for step in ZCH:
            pltpu.async_copy(
                rows.at[pl.ds(0, step)],
                acc.at[pl.ds(base_r + off, step)], semg0)
            off += step
        for step in ZCH:
            pltpu.make_async_copy(
                rows.at[pl.ds(0, step)],
                acc.at[pl.ds(base_r, step)], semg0).wait()

        pltpu.sync_copy(src_hbm.at[wid, 0], sidx.at[0])
        pltpu.sync_copy(dst_hbm.at[wid, 0], didx.at[0])
        plsc.subcore_barrier()

        def chunk(cc, _):
            b = lax.rem(cc, 2)
            blk = cc // CB
            j = lax.rem(cc, CB)
            bb = lax.rem(blk, 2)
            jp = lax.rem(cc - 1, CB)
            bp = lax.rem((cc - 1) // CB, 2)
            rg = pl.multiple_of(b * K, 8)
            rs = pl.multiple_of((1 - b) * K, 8)

            @pl.when(cc >= 2)
            def _():
                pltpu.make_async_copy(
                    rows.at[pl.ds(0, K)], acc.at[didx.at[0, 0]], sems).wait()

            @pl.when(jnp.logical_and(j == 2, blk + 1 < NB))
            def _():
                pltpu.async_copy(
                    src_hbm.at[wid, blk + 1], sidx.at[1 - bb], semi)
                pltpu.async_copy(
                    dst_hbm.at[wid, blk + 1], didx.at[1 - bb], semi)

            @pl.when(jnp.logical_and(cc < NCH, b == 0))
            def _():
                pltpu.async_copy(
                    h_hbm.at[sidx.at[bb, j]], rows.at[pl.ds(rg, K)], semg0)

            @pl.when(jnp.logical_and(cc < NCH, b == 1))
            def _():
                pltpu.async_copy(
                    h_hbm.at[sidx.at[bb, j]], rows.at[pl.ds(rg, K)], semg1)

            @pl.when(jnp.logical_and(cc >= 1, b == 1))
            def _():
                pltpu.make_async_copy(
                    h_hbm.at[sidx.at[0, 0]], rows.at[pl.ds(0, K)],
                    semg0).wait()

            @pl.when(jnp.logical_and(cc >= 1, b == 0))
            def _():
                pltpu.make_async_copy(
                    h_hbm.at[sidx.at[0, 0]], rows.at[pl.ds(0, K)],
                    semg1).wait()

            @pl.when(cc >= 1)
            def _():
                pltpu.async_copy(
                    rows.at[pl.ds(rs, K)], acc.at[didx.at[bp, jp]],
                    sems, add=True)

            @pl.when(jnp.logical_and(j == CB - 1, blk + 1 < NB))
            def _():
                pltpu.make_async_copy(
                    src_hbm.at[wid, 0], sidx.at[0], semi).wait()
                pltpu.make_async_copy(
                    dst_hbm.at[wid, 0], didx.at[0], semi).wait()

            return 0
        lax.fori_loop(0, NCH + 1, chunk, 0)
        pltpu.make_async_copy(
            rows.at[pl.ds(0, K)], acc.at[didx.at[0, 0]], sems).wait()
        plsc.subcore_barrier()

        @pl.when(c == 0)
        def _():
            off2 = 0
            for step in ZCH:
                sl = pl.ds(base_r + off2, step)
                pltpu.async_copy(acc.at[sl], agg0.at[sl], semg0)
                off2 += step
            for step in ZCH:
                pltpu.make_async_copy(
                    acc.at[pl.ds(base_r, step)],
                    agg0.at[pl.ds(base_r, step)], semg0).wait()

        @pl.when(c == 1)
        def _():
            off2 = 0
            for step in ZCH:
                sl = pl.ds(base_r + off2, step)
                pltpu.async_copy(acc.at[sl], agg1.at[sl], semg0)
                off2 += step
            for step in ZCH:
                pltpu.make_async_copy(
                    acc.at[pl.ds(base_r, step)],
                    agg1.at[pl.ds(base_r, step)], semg0).wait()

    return agg_kernel


def _make_prescale(pad_cnt):
    def _prescale_body(x_ref, a_ref, b_ref, o_ref):
        deg = a_ref[...] + b_ref[...]
        corr = jnp.where(
            jnp.logical_and(
                lax.broadcasted_iota(jnp.int32, deg.shape, 0) == 0,
                pl.program_id(0) == 0),
            jnp.float32(pad_cnt), jnp.float32(0.0))
        norm = lax.rsqrt(jnp.maximum(deg - corr, 1.0))
        o_ref[...] = x_ref[...] * norm
    return _prescale_body


def _final_body(a0_ref, a1_ref, d0_ref, d1_ref, w_ref, o_ref):
    agg = a0_ref[...] + a1_ref[...]
    deg = d0_ref[...] + d1_ref[...]
    norm = lax.rsqrt(jnp.maximum(deg, 1.0))
    o_ref[...] = jnp.dot(agg * norm, w_ref[...],
                         preferred_element_type=jnp.float32)


def kernel(x, edge_index, W):
    N, D = x.shape
    E = edge_index.shape[1]
    N_PAD = ((N + 1 + 127) // 128) * 128
    N_PADH = ((N + 1 + 255) // 256) * 256
    EB = NW * CB * K
    E_PAD = ((E + EB - 1) // EB) * EB
    pad_cnt = E_PAD - E
    assert N % BR == 0
    grid_n = N // BR

    src = edge_index[0]
    dst = edge_index[1]
    if pad_cnt:
        src = jnp.concatenate([src, jnp.zeros((pad_cnt,), jnp.int32)])
        pad_dst = N + jnp.arange(pad_cnt, dtype=jnp.int32) % (N_PAD - N)
        dst = jnp.concatenate([dst, pad_dst])
    src = src.reshape(NW, E_PAD // (NW * CB * K), CB, K)
    dst = dst.reshape(NW, E_PAD // (NW * CB * K), CB, K)

    hs0, hd0, hs1, hd1 = _build_hist(E_PAD, N_PADH)(src, dst)

    h = pl.pallas_call(
        _make_prescale(pad_cnt),
        grid=(grid_n,),
        in_specs=[
            pl.BlockSpec((BR, D), lambda i: (i, 0)),
            pl.BlockSpec((BR, 1), lambda i: (i, 0)),
            pl.BlockSpec((BR, 1), lambda i: (i, 0)),
        ],
        out_specs=pl.BlockSpec((BR, D), lambda i: (i, 0)),
        out_shape=jax.ShapeDtypeStruct((N, D), jnp.float32),
    )(x, hs0.reshape(-1, 1), hs1.reshape(-1, 1))

    agg0, agg1 = _build_agg(E_PAD, N, N_PAD, D)(h, src, dst)

    out = pl.pallas_call(
        _final_body,
        grid=(grid_n,),
        in_specs=[
            pl.BlockSpec((BR, D), lambda i: (i, 0)),
            pl.BlockSpec((BR, D), lambda i: (i, 0)),
            pl.BlockSpec((BR, 1), lambda i: (i, 0)),
            pl.BlockSpec((BR, 1), lambda i: (i, 0)),
            pl.BlockSpec((D, D), lambda i: (0, 0)),
        ],
        out_specs=pl.BlockSpec((BR, D), lambda i: (i, 0)),
        out_shape=jax.ShapeDtypeStruct((N, D), jnp.float32),
    )(agg0, agg1, hd0.reshape(-1, 1), hd1.reshape(-1, 1), W)

    return out


# speedup vs baseline: 1.0088x; 1.0088x over previous
import functools

import jax
import jax.numpy as jnp
from jax import lax
from jax.experimental import pallas as pl
from jax.experimental.pallas import tpu as pltpu
from jax.experimental.pallas import tpu_sc as plsc

NC = 2
NS = 16
NW = NC * NS
LANES = 16
K = 128
CB = 8
BR = 400


def _mesh():
    return plsc.VectorSubcoreMesh(core_axis_name="c", subcore_axis_name="s")


@functools.lru_cache(maxsize=None)
def _build_hist(E_PAD, N_PAD):
    EPW = E_PAD // NW
    NCH = EPW // K
    NB = NCH // CB
    ZH = N_PAD // NS
    ZF = ((ZH + LANES - 1) // LANES) * LANES
    FIRE = 4
    f32 = jnp.float32
    sds = jax.ShapeDtypeStruct

    @functools.partial(
        pl.kernel,
        out_type=(sds((N_PAD,), f32),) * 4,
        mesh=_mesh(),
        scratch_types=[
            pltpu.VMEM((NB, CB, K), jnp.int32),
            pltpu.VMEM((NB, CB, K), jnp.int32),
            pltpu.VMEM((K,), f32),
            pltpu.VMEM((ZF,), f32),
            pltpu.VMEM_SHARED((N_PAD,), f32),
            pltpu.VMEM_SHARED((N_PAD,), f32),
            pltpu.SemaphoreType.DMA,
        ],
    )
    def hist_kernel(src_hbm, dst_hbm, hs0, hd0, hs1, hd1,
                    sidx, didx, ones_v, zb, hist_s, hist_d, sem):
        c = lax.axis_index("c")
        s = lax.axis_index("s")
        wid = s * NC + c

        def fill_z(i, _):
            zb[pl.ds(i * LANES, LANES)] = jnp.zeros((LANES,), f32)
            return 0
        lax.fori_loop(0, ZF // LANES, fill_z, 0)

        def fill_o(i, _):
            ones_v[pl.ds(i * LANES, LANES)] = jnp.ones((LANES,), f32)
            return 0
        lax.fori_loop(0, K // LANES, fill_o, 0)

        r0 = pl.multiple_of(s * ZH, 8)
        pltpu.sync_copy(zb.at[pl.ds(0, ZH)], hist_s.at[pl.ds(r0, ZH)])
        pltpu.sync_copy(zb.at[pl.ds(0, ZH)], hist_d.at[pl.ds(r0, ZH)])

        pltpu.sync_copy(src_hbm.at[wid], sidx)
        pltpu.sync_copy(dst_hbm.at[wid], didx)
        plsc.subcore_barrier()

        def fire_block(ob, _):
            for j in range(CB):
                pltpu.async_copy(
                    ones_v, hist_s.at[sidx.at[ob, j]], sem, add=True)
                pltpu.async_copy(
                    ones_v, hist_d.at[didx.at[ob, j]], sem, add=True)
                if j % FIRE == FIRE - 1:
                    for _k in range(2 * FIRE):
                        pltpu.make_async_copy(
                            ones_v, hist_s.at[sidx.at[0, 0]], sem).wait()
            return 0
        lax.fori_loop(0, NB, fire_block, 0)
        plsc.subcore_barrier()

        @pl.when(c == 0)
        def _():
            pltpu.sync_copy(hist_s.at[pl.ds(r0, ZH)], hs0.at[pl.ds(r0, ZH)])
            pltpu.sync_copy(hist_d.at[pl.ds(r0, ZH)], hd0.at[pl.ds(r0, ZH)])

        @pl.when(c == 1)
        def _():
            pltpu.sync_copy(hist_s.at[pl.ds(r0, ZH)], hs1.at[pl.ds(r0, ZH)])
            pltpu.sync_copy(hist_d.at[pl.ds(r0, ZH)], hd1.at[pl.ds(r0, ZH)])

    return hist_kernel


@functools.lru_cache(maxsize=None)
def _build_agg(E_PAD, N, N_PAD, D):
    EPW = E_PAD // NW
    NCH = EPW // K
    NB = NCH // CB
    RPT = N_PAD // NS
    ZOFF = list(range(0, RPT - K + 1, K))
    if ZOFF[-1] != RPT - K:
        ZOFF.append(RPT - K)
    f32 = jnp.float32
    sds = jax.ShapeDtypeStruct

    @functools.partial(
        pl.kernel,
        out_type=(sds((N_PAD, D), f32), sds((N_PAD, D), f32)),
        mesh=_mesh(),
        scratch_types=[
            pltpu.VMEM((2, CB, K), jnp.int32),
            pltpu.VMEM((2, CB, K), jnp.int32),
            pltpu.VMEM((2, K, D), f32),
            pltpu.VMEM_SHARED((N_PAD, D), f32),
            pltpu.SemaphoreType.DMA,
            pltpu.SemaphoreType.DMA,
            pltpu.SemaphoreType.DMA,
            pltpu.SemaphoreType.DMA,
        ],
    )
    def agg_kernel(h_hbm, src_hbm, dst_hbm, agg0, agg1,
                   sidx, didx, rows, acc, semg0, semg1, sems, semi):
        c = lax.axis_index("c")
        s = lax.axis_index("s")
        wid = s * NC + c

        def fill_z(r, _):
            for jj in range(D // LANES):
                rows[0, r, pl.ds(jj * LANES, LANES)] = jnp.zeros(
                    (LANES,), f32)
            return 0
        lax.fori_loop(0, K, fill_z, 0)

        base_r = s * RPT
        for off in ZOFF:
            pltpu.async_copy(
                rows.at[0], acc.at[pl.ds(base_r + off, K)], semg0)
        for off in ZOFF:
            pltpu.make_async_copy(
                rows.at[0], acc.at[pl.ds(base_r, K)], semg0).wait()

        pltpu.sync_copy(src_hbm.at[wid, 0], sidx.at[0])
        pltpu.sync_copy(dst_hbm.at[wid, 0], didx.at[0])
        plsc.subcore_barrier()

        def chunk(cc, _):
            b = lax.rem(cc, 2)
            blk = cc // CB
            j = lax.rem(cc, CB)
            bb = lax.rem(blk, 2)
            jp = lax.rem(cc - 1, CB)
            bp = lax.rem((cc - 1) // CB, 2)

            @pl.when(cc >= 2)
            def _():
                pltpu.make_async_copy(
                    rows.at[0], acc.at[didx.at[0, 0]], sems).wait()

            @pl.when(jnp.logical_and(j == 2, blk + 1 < NB))
            def _():
                pltpu.async_copy(
                    src_hbm.at[wid, blk + 1], sidx.at[1 - bb], semi)
                pltpu.async_copy(
                    dst_hbm.at[wid, blk + 1], didx.at[1 - bb], semi)

            @pl.when(jnp.logical_and(cc < NCH, b == 0))
            def _():
                pltpu.async_copy(
                    h_hbm.at[sidx.at[bb, j]], rows.at[b], semg0)

            @pl.when(jnp.logical_and(cc < NCH, b == 1))
            def _():
                pltpu.async_copy(
                    h_hbm.at[sidx.at[bb, j]], rows.at[b], semg1)

            @pl.when(jnp.logical_and(cc >= 1, b == 1))
            def _():
                pltpu.make_async_copy(
                    h_hbm.at[sidx.at[0, 0]], rows.at[0], semg0).wait()

            @pl.when(jnp.logical_and(cc >= 1, b == 0))
            def _():
                pltpu.make_async_copy(
                    h_hbm.at[sidx.at[0, 0]], rows.at[0], semg1).wait()

            @pl.when(cc >= 1)
            def _():
                pltpu.async_copy(
                    rows.at[1 - b], acc.at[didx.at[bp, jp]],
                    sems, add=True)

            @pl.when(jnp.logical_and(j == CB - 1, blk + 1 < NB))
            def _():
                pltpu.make_async_copy(
                    src_hbm.at[wid, 0], sidx.at[0], semi).wait()
                pltpu.make_async_copy(
                    dst_hbm.at[wid, 0], didx.at[0], semi).wait()

            return 0
        lax.fori_loop(0, NCH + 1, chunk, 0)
        pltpu.make_async_copy(
            rows.at[0], acc.at[didx.at[0, 0]], sems).wait()
        plsc.subcore_barrier()

        @pl.when(c == 0)
        def _():
            for off in ZOFF:
                sl = pl.ds(base_r + off, K)
                pltpu.async_copy(acc.at[sl], agg0.at[sl], semg0)
            for off in ZOFF:
                pltpu.make_async_copy(
                    acc.at[pl.ds(base_r, K)],
                    agg0.at[pl.ds(base_r, K)], semg0).wait()

        @pl.when(c == 1)
        def _():
            for off in ZOFF:
                sl = pl.ds(base_r + off, K)
                pltpu.async_copy(acc.at[sl], agg1.at[sl], semg0)
            for off in ZOFF:
                pltpu.make_async_copy(
                    acc.at[pl.ds(base_r, K)],
                    agg1.at[pl.ds(base_r, K)], semg0).wait()

    return agg_kernel


def _make_prescale(pad_cnt):
    def _prescale_body(x_ref, a_ref, b_ref, o_ref):
        deg = a_ref[...] + b_ref[...]
        corr = jnp.where(
            jnp.logical_and(
                lax.broadcasted_iota(jnp.int32, deg.shape, 0) == 0,
                pl.program_id(0) == 0),
            jnp.float32(pad_cnt), jnp.float32(0.0))
        norm = lax.rsqrt(jnp.maximum(deg - corr, 1.0))
        o_ref[...] = x_ref[...] * norm
    return _prescale_body


def _final_body(a0_ref, a1_ref, d0_ref, d1_ref, w_ref, o_ref):
    agg = a0_ref[...] + a1_ref[...]
    deg = d0_ref[...] + d1_ref[...]
    norm = lax.rsqrt(jnp.maximum(deg, 1.0))
    o_ref[...] = jnp.dot(agg * norm, w_ref[...],
                         preferred_element_type=jnp.float32)


def kernel(x, edge_index, W):
    N, D = x.shape
    E = edge_index.shape[1]
    N_PAD = ((N + 1 + 127) // 128) * 128
    N_PADH = ((N + 1 + 255) // 256) * 256
    EB = NW * CB * K
    E_PAD = ((E + EB - 1) // EB) * EB
    pad_cnt = E_PAD - E
    assert N % BR == 0
    grid_n = N // BR

    src = edge_index[0]
    dst = edge_index[1]
    if pad_cnt:
        src = jnp.concatenate([src, jnp.zeros((pad_cnt,), jnp.int32)])
        pad_dst = N + jnp.arange(pad_cnt, dtype=jnp.int32) % (N_PAD - N)
        dst = jnp.concatenate([dst, pad_dst])
    src = src.reshape(NW, E_PAD // (NW * CB * K), CB, K)
    dst = dst.reshape(NW, E_PAD // (NW * CB * K), CB, K)

    hs0, hd0, hs1, hd1 = _build_hist(E_PAD, N_PADH)(src, dst)

    h = pl.pallas_call(
        _make_prescale(pad_cnt),
        grid=(grid_n,),
        in_specs=[
            pl.BlockSpec((BR, D), lambda i: (i, 0)),
            pl.BlockSpec((BR, 1), lambda i: (i, 0)),
            pl.BlockSpec((BR, 1), lambda i: (i, 0)),
        ],
        out_specs=pl.BlockSpec((BR, D), lambda i: (i, 0)),
        out_shape=jax.ShapeDtypeStruct((N, D), jnp.float32),
    )(x, hs0.reshape(-1, 1), hs1.reshape(-1, 1))

    agg0, agg1 = _build_agg(E_PAD, N, N_PAD, D)(h, src, dst)

    out = pl.pallas_call(
        _final_body,
        grid=(grid_n,),
        in_specs=[
            pl.BlockSpec((BR, D), lambda i: (i, 0)),
            pl.BlockSpec((BR, D), lambda i: (i, 0)),
            pl.BlockSpec((BR, 1), lambda i: (i, 0)),
            pl.BlockSpec((BR, 1), lambda i: (i, 0)),
            pl.BlockSpec((D, D), lambda i: (0, 0)),
        ],
        out_specs=pl.BlockSpec((BR, D), lambda i: (i, 0)),
        out_shape=jax.ShapeDtypeStruct((N, D), jnp.float32),
    )(agg0, agg1, hd0.reshape(-1, 1), hd1.reshape(-1, 1), W)

    return out


# speedup vs baseline: 2.8843x; 2.8591x over previous
import functools

import jax
import jax.numpy as jnp
from jax import lax
from jax.experimental import pallas as pl
from jax.experimental.pallas import tpu as pltpu
from jax.experimental.pallas import tpu_sc as plsc

NC = 2
NS = 16
NW = NC * NS
LANES = 16
K = 128
CB = 8
BR = 400


def _mesh():
    return plsc.VectorSubcoreMesh(core_axis_name="c", subcore_axis_name="s")


@functools.lru_cache(maxsize=None)
def _build_hist(E_PAD, N_PAD):
    EPW = E_PAD // NW
    NCH = EPW // K
    NB = NCH // CB
    ZH = N_PAD // NS
    ZF = ((ZH + LANES - 1) // LANES) * LANES
    FIRE = 4
    f32 = jnp.float32
    sds = jax.ShapeDtypeStruct

    @functools.partial(
        pl.kernel,
        out_type=(sds((N_PAD,), f32),) * 4,
        mesh=_mesh(),
        scratch_types=[
            pltpu.VMEM((NB, CB, K), jnp.int32),
            pltpu.VMEM((NB, CB, K), jnp.int32),
            pltpu.VMEM((K,), f32),
            pltpu.VMEM((ZF,), f32),
            pltpu.VMEM_SHARED((N_PAD,), f32),
            pltpu.VMEM_SHARED((N_PAD,), f32),
            pltpu.SemaphoreType.DMA,
        ],
    )
    def hist_kernel(src_hbm, dst_hbm, hs0, hd0, hs1, hd1,
                    sidx, didx, ones_v, zb, hist_s, hist_d, sem):
        c = lax.axis_index("c")
        s = lax.axis_index("s")
        wid = s * NC + c

        def fill_z(i, _):
            zb[pl.ds(i * LANES, LANES)] = jnp.zeros((LANES,), f32)
            return 0
        lax.fori_loop(0, ZF // LANES, fill_z, 0)

        def fill_o(i, _):
            ones_v[pl.ds(i * LANES, LANES)] = jnp.ones((LANES,), f32)
            return 0
        lax.fori_loop(0, K // LANES, fill_o, 0)

        r0 = pl.multiple_of(s * ZH, 8)
        pltpu.sync_copy(zb.at[pl.ds(0, ZH)], hist_s.at[pl.ds(r0, ZH)])
        pltpu.sync_copy(zb.at[pl.ds(0, ZH)], hist_d.at[pl.ds(r0, ZH)])

        pltpu.sync_copy(src_hbm.at[wid], sidx)
        pltpu.sync_copy(dst_hbm.at[wid], didx)
        plsc.subcore_barrier()

        def fire_block(ob, _):
            for j in range(CB):
                pltpu.async_copy(
                    ones_v, hist_s.at[sidx.at[ob, j]], sem, add=True)
                pltpu.async_copy(
                    ones_v, hist_d.at[didx.at[ob, j]], sem, add=True)
                if j % FIRE == FIRE - 1:
                    for _k in range(2 * FIRE):
                        pltpu.make_async_copy(
                            ones_v, hist_s.at[sidx.at[0, 0]], sem).wait()
            return 0
        lax.fori_loop(0, NB, fire_block, 0)
        plsc.subcore_barrier()

        @pl.when(c == 0)
        def _():
            pltpu.sync_copy(hist_s.at[pl.ds(r0, ZH)], hs0.at[pl.ds(r0, ZH)])
            pltpu.sync_copy(hist_d.at[pl.ds(r0, ZH)], hd0.at[pl.ds(r0, ZH)])

        @pl.when(c == 1)
        def _():
            pltpu.sync_copy(hist_s.at[pl.ds(r0, ZH)], hs1.at[pl.ds(r0, ZH)])
            pltpu.sync_copy(hist_d.at[pl.ds(r0, ZH)], hd1.at[pl.ds(r0, ZH)])

    return hist_kernel


@functools.lru_cache(maxsize=None)
def _build_agg(E_PAD, N, N_PAD, D):
    EPW = E_PAD // NW
    NCH = EPW // K
    NB = NCH // CB
    RPT = N_PAD // NS
    ZOFF = list(range(0, RPT - K + 1, K))
    if ZOFF[-1] != RPT - K:
        ZOFF.append(RPT - K)
    f32 = jnp.float32
    sds = jax.ShapeDtypeStruct

    @functools.partial(
        pl.kernel,
        out_type=(sds((N_PAD, D), f32), sds((N_PAD, D), f32)),
        mesh=_mesh(),
        scratch_types=[
            pltpu.VMEM((2, CB, K), jnp.int32),
            pltpu.VMEM((2, CB, K), jnp.int32),
            pltpu.VMEM((2, K, D), f32),
            pltpu.VMEM_SHARED((N_PAD, D), f32),
            pltpu.SemaphoreType.DMA,
            pltpu.SemaphoreType.DMA,
            pltpu.SemaphoreType.DMA,
            pltpu.SemaphoreType.DMA,
        ],
    )
    def agg_kernel(h_hbm, src_hbm, dst_hbm, agg0, agg1,
                   sidx, didx, rows, acc, semg0, semg1, sems, semi):
        c = lax.axis_index("c")
        s = lax.axis_index("s")
        wid = s * NC + c

        def fill_z(r, _):
            for jj in range(D // LANES):
                rows[0, r, pl.ds(jj * LANES, LANES)] = jnp.zeros(
                    (LANES,), f32)
            return 0
        lax.fori_loop(0, K, fill_z, 0)

        base_r = s * RPT
        for off in ZOFF:
            pltpu.async_copy(
                rows.at[0], acc.at[pl.ds(base_r + off, K)], semg0)
        for off in ZOFF:
            pltpu.make_async_copy(
                rows.at[0], acc.at[pl.ds(base_r, K)], semg0).wait()

        pltpu.sync_copy(src_hbm.at[wid, 0], sidx.at[0])
        pltpu.sync_copy(dst_hbm.at[wid, 0], didx.at[0])
        plsc.subcore_barrier()

        def chunk(cc, _):
            b = lax.rem(cc, 2)
            blk = cc // CB
            j = lax.rem(cc, CB)
            bb = lax.rem(blk, 2)
            jp = lax.rem(cc - 1, CB)
            bp = lax.rem((cc - 1) // CB, 2)

            @pl.when(cc >= 2)
            def _():
                pltpu.make_async_copy(
                    rows.at[0], acc.at[didx.at[0, 0]], sems).wait()

            @pl.when(jnp.logical_and(j == 2, blk + 1 < NB))
            def _():
                pltpu.async_copy(
                    src_hbm.at[wid, blk + 1], sidx.at[1 - bb], semi)
                pltpu.async_copy(
                    dst_hbm.at[wid, blk + 1], didx.at[1 - bb], semi)

            @pl.when(jnp.logical_and(cc < NCH, b == 0))
            def _():
                pltpu.async_copy(
                    h_hbm.at[sidx.at[bb, j]], rows.at[b], semg0)

            @pl.when(jnp.logical_and(cc < NCH, b == 1))
            def _():
                pltpu.async_copy(
                    h_hbm.at[sidx.at[bb, j]], rows.at[b], semg1)

            @pl.when(jnp.logical_and(cc >= 1, b == 1))
            def _():
                pltpu.make_async_copy(
                    h_hbm.at[sidx.at[0, 0]], rows.at[0], semg0).wait()

            @pl.when(jnp.logical_and(cc >= 1, b == 0))
            def _():
                pltpu.make_async_copy(
                    h_hbm.at[sidx.at[0, 0]], rows.at[0], semg1).wait()

            @pl.when(cc >= 1)
            def _():
                pltpu.async_copy(
                    rows.at[1 - b], acc.at[didx.at[bp, jp]],
                    sems, add=True)

            @pl.when(jnp.logical_and(j == CB - 1, blk + 1 < NB))
            def _():
                pltpu.make_async_copy(
                    src_hbm.at[wid, 0], sidx.at[0], semi).wait()
                pltpu.make_async_copy(
                    dst_hbm.at[wid, 0], didx.at[0], semi).wait()

            return 0
        lax.fori_loop(0, NCH + 1, chunk, 0)
        pltpu.make_async_copy(
            rows.at[0], acc.at[didx.at[0, 0]], sems).wait()
        plsc.subcore_barrier()

        @pl.when(c == 0)
        def _():
            for off in ZOFF:
                sl = pl.ds(base_r + off, K)
                pltpu.async_copy(acc.at[sl], agg0.at[sl], semg0)
            for off in ZOFF:
                pltpu.make_async_copy(
                    acc.at[pl.ds(base_r, K)],
                    agg0.at[pl.ds(base_r, K)], semg0).wait()

        @pl.when(c == 1)
        def _():
            for off in ZOFF:
                sl = pl.ds(base_r + off, K)
                pltpu.async_copy(acc.at[sl], agg1.at[sl], semg0)
            for off in ZOFF:
                pltpu.make_async_copy(
                    acc.at[pl.ds(base_r, K)],
                    agg1.at[pl.ds(base_r, K)], semg0).wait()

    return agg_kernel


def _make_prescale(pad_cnt):
    full, rem = pad_cnt // K, pad_cnt % K

    def _prescale_body(x_ref, a_ref, b_ref, o_ref):
        deg = a_ref[...] + b_ref[...]
        row = lax.broadcasted_iota(jnp.int32, deg.shape, 0)
        on_first = pl.program_id(0) == 0
        corr = (jnp.where(jnp.logical_and(row < K, on_first), full, 0)
                + jnp.where(jnp.logical_and(row < rem, on_first), 1, 0))
        norm = lax.rsqrt(jnp.maximum(deg - corr.astype(jnp.float32), 1.0))
        o_ref[...] = x_ref[...] * norm
    return _prescale_body


def _final_body(a0_ref, a1_ref, d0_ref, d1_ref, w_ref, o_ref):
    agg = a0_ref[...] + a1_ref[...]
    deg = d0_ref[...] + d1_ref[...]
    norm = lax.rsqrt(jnp.maximum(deg, 1.0))
    o_ref[...] = jnp.dot(agg * norm, w_ref[...],
                         preferred_element_type=jnp.float32)


def kernel(x, edge_index, W):
    N, D = x.shape
    E = edge_index.shape[1]
    N_PAD = ((N + 1 + 127) // 128) * 128
    N_PADH = ((N + 1 + 255) // 256) * 256
    EB = NW * CB * K
    E_PAD = ((E + EB - 1) // EB) * EB
    pad_cnt = E_PAD - E
    assert N % BR == 0
    grid_n = N // BR

    src = edge_index[0]
    dst = edge_index[1]
    if pad_cnt:
        pad_src = jnp.arange(pad_cnt, dtype=jnp.int32) % K
        pad_dst = N + jnp.arange(pad_cnt, dtype=jnp.int32) % (N_PAD - N)
        src = jnp.concatenate([src, pad_src])
        dst = jnp.concatenate([dst, pad_dst])
    src = src.reshape(NW, E_PAD // (NW * CB * K), CB, K)
    dst = dst.reshape(NW, E_PAD // (NW * CB * K), CB, K)

    hs0, hd0, hs1, hd1 = _build_hist(E_PAD, N_PADH)(src, dst)

    h = pl.pallas_call(
        _make_prescale(pad_cnt),
        grid=(grid_n,),
        in_specs=[
            pl.BlockSpec((BR, D), lambda i: (i, 0)),
            pl.BlockSpec((BR, 1), lambda i: (i, 0)),
            pl.BlockSpec((BR, 1), lambda i: (i, 0)),
        ],
        out_specs=pl.BlockSpec((BR, D), lambda i: (i, 0)),
        out_shape=jax.ShapeDtypeStruct((N, D), jnp.float32),
    )(x, hs0.reshape(-1, 1), hs1.reshape(-1, 1))

    agg0, agg1 = _build_agg(E_PAD, N, N_PAD, D)(h, src, dst)

    out = pl.pallas_call(
        _final_body,
        grid=(grid_n,),
        in_specs=[
            pl.BlockSpec((BR, D), lambda i: (i, 0)),
            pl.BlockSpec((BR, D), lambda i: (i, 0)),
            pl.BlockSpec((BR, 1), lambda i: (i, 0)),
            pl.BlockSpec((BR, 1), lambda i: (i, 0)),
            pl.BlockSpec((D, D), lambda i: (0, 0)),
        ],
        out_specs=pl.BlockSpec((BR, D), lambda i: (i, 0)),
        out_shape=jax.ShapeDtypeStruct((N, D), jnp.float32),
    )(agg0, agg1, hd0.reshape(-1, 1), hd1.reshape(-1, 1), W)

    return out


# speedup vs baseline: 3.2188x; 1.1160x over previous
import functools

import numpy as np

import jax
import jax.numpy as jnp
from jax import lax
from jax.experimental import pallas as pl
from jax.experimental.pallas import tpu as pltpu
from jax.experimental.pallas import tpu_sc as plsc

NC = 2
NS = 16
NW = NC * NS
LANES = 16
K = 128
CB = 8
BR = 2000


def _mesh():
    return plsc.VectorSubcoreMesh(core_axis_name="c", subcore_axis_name="s")


@functools.lru_cache(maxsize=None)
def _build_hist(E_PAD, N_PAD):
    EPW = E_PAD // NW
    NCH = EPW // K
    NB = NCH // CB
    ZH = N_PAD // NS
    ZF = ((ZH + LANES - 1) // LANES) * LANES
    FIRE = 4
    f32 = jnp.float32
    sds = jax.ShapeDtypeStruct

    @functools.partial(
        pl.kernel,
        out_type=(sds((N_PAD,), f32),) * 4,
        mesh=_mesh(),
        scratch_types=[
            pltpu.VMEM((NB, CB, K), jnp.int32),
            pltpu.VMEM((NB, CB, K), jnp.int32),
            pltpu.VMEM((K,), f32),
            pltpu.VMEM((ZF,), f32),
            pltpu.VMEM_SHARED((N_PAD,), f32),
            pltpu.VMEM_SHARED((N_PAD,), f32),
            pltpu.SemaphoreType.DMA,
        ],
    )
    def hist_kernel(src_hbm, dst_hbm, hs0, hd0, hs1, hd1,
                    sidx, didx, ones_v, zb, hist_s, hist_d, sem):
        c = lax.axis_index("c")
        s = lax.axis_index("s")
        wid = s * NC + c

        def fill_z(i, _):
            zb[pl.ds(i * LANES, LANES)] = jnp.zeros((LANES,), f32)
            return 0
        lax.fori_loop(0, ZF // LANES, fill_z, 0)

        def fill_o(i, _):
            ones_v[pl.ds(i * LANES, LANES)] = jnp.ones((LANES,), f32)
            return 0
        lax.fori_loop(0, K // LANES, fill_o, 0)

        r0 = pl.multiple_of(s * ZH, 8)
        pltpu.sync_copy(zb.at[pl.ds(0, ZH)], hist_s.at[pl.ds(r0, ZH)])
        pltpu.sync_copy(zb.at[pl.ds(0, ZH)], hist_d.at[pl.ds(r0, ZH)])

        pltpu.sync_copy(src_hbm.at[wid], sidx)
        pltpu.sync_copy(dst_hbm.at[wid], didx)
        plsc.subcore_barrier()

        def fire_block(ob, _):
            for j in range(CB):
                pltpu.async_copy(
                    ones_v, hist_s.at[sidx.at[ob, j]], sem, add=True)
                pltpu.async_copy(
                    ones_v, hist_d.at[didx.at[ob, j]], sem, add=True)
                if j % FIRE == FIRE - 1:
                    for _k in range(2 * FIRE):
                        pltpu.make_async_copy(
                            ones_v, hist_s.at[sidx.at[0, 0]], sem).wait()
            return 0
        lax.fori_loop(0, NB, fire_block, 0)
        plsc.subcore_barrier()

        @pl.when(c == 0)
        def _():
            pltpu.sync_copy(hist_s.at[pl.ds(r0, ZH)], hs0.at[pl.ds(r0, ZH)])
            pltpu.sync_copy(hist_d.at[pl.ds(r0, ZH)], hd0.at[pl.ds(r0, ZH)])

        @pl.when(c == 1)
        def _():
            pltpu.sync_copy(hist_s.at[pl.ds(r0, ZH)], hs1.at[pl.ds(r0, ZH)])
            pltpu.sync_copy(hist_d.at[pl.ds(r0, ZH)], hd1.at[pl.ds(r0, ZH)])

    return hist_kernel


@functools.lru_cache(maxsize=None)
def _build_agg(E_PAD, N, N_PAD, D):
    EPW = E_PAD // NW
    NCH = EPW // K
    NB = NCH // CB
    RPT = N_PAD // NS
    ZOFF = list(range(0, RPT - K + 1, K))
    if ZOFF[-1] != RPT - K:
        ZOFF.append(RPT - K)
    f32 = jnp.float32
    sds = jax.ShapeDtypeStruct

    @functools.partial(
        pl.kernel,
        out_type=(sds((N_PAD, D), f32), sds((N_PAD, D), f32)),
        mesh=_mesh(),
        scratch_types=[
            pltpu.VMEM((2, CB, K), jnp.int32),
            pltpu.VMEM((2, CB, K), jnp.int32),
            pltpu.VMEM((2, K, D), f32),
            pltpu.VMEM_SHARED((N_PAD, D), f32),
            pltpu.SemaphoreType.DMA,
            pltpu.SemaphoreType.DMA,
            pltpu.SemaphoreType.DMA,
            pltpu.SemaphoreType.DMA,
        ],
    )
    def agg_kernel(h_hbm, src_hbm, dst_hbm, agg0, agg1,
                   sidx, didx, rows, acc, semg0, semg1, sems, semi):
        c = lax.axis_index("c")
        s = lax.axis_index("s")
        wid = s * NC + c

        def fill_z(r, _):
            for jj in range(D // LANES):
                rows[0, r, pl.ds(jj * LANES, LANES)] = jnp.zeros(
                    (LANES,), f32)
            return 0
        lax.fori_loop(0, K, fill_z, 0)

        base_r = s * RPT
        for off in ZOFF:
            pltpu.async_copy(
                rows.at[0], acc.at[pl.ds(base_r + off, K)], semg0)
        for off in ZOFF:
            pltpu.make_async_copy(
                rows.at[0], acc.at[pl.ds(base_r, K)], semg0).wait()

        pltpu.sync_copy(src_hbm.at[wid, 0], sidx.at[0])
        pltpu.sync_copy(dst_hbm.at[wid, 0], didx.at[0])
        plsc.subcore_barrier()

        def chunk(cc, _):
            b = lax.rem(cc, 2)
            blk = cc // CB
            j = lax.rem(cc, CB)
            bb = lax.rem(blk, 2)
            jp = lax.rem(cc - 1, CB)
            bp = lax.rem((cc - 1) // CB, 2)

            @pl.when(cc >= 2)
            def _():
                pltpu.make_async_copy(
                    rows.at[0], acc.at[didx.at[0, 0]], sems).wait()

            @pl.when(jnp.logical_and(j == 2, blk + 1 < NB))
            def _():
                pltpu.async_copy(
                    src_hbm.at[wid, blk + 1], sidx.at[1 - bb], semi)
                pltpu.async_copy(
                    dst_hbm.at[wid, blk + 1], didx.at[1 - bb], semi)

            @pl.when(jnp.logical_and(cc < NCH, b == 0))
            def _():
                pltpu.async_copy(
                    h_hbm.at[sidx.at[bb, j]], rows.at[b], semg0)

            @pl.when(jnp.logical_and(cc < NCH, b == 1))
            def _():
                pltpu.async_copy(
                    h_hbm.at[sidx.at[bb, j]], rows.at[b], semg1)

            @pl.when(jnp.logical_and(cc >= 1, b == 1))
            def _():
                pltpu.make_async_copy(
                    h_hbm.at[sidx.at[0, 0]], rows.at[0], semg0).wait()

            @pl.when(jnp.logical_and(cc >= 1, b == 0))
            def _():
                pltpu.make_async_copy(
                    h_hbm.at[sidx.at[0, 0]], rows.at[0], semg1).wait()

            @pl.when(cc >= 1)
            def _():
                pltpu.async_copy(
                    rows.at[1 - b], acc.at[didx.at[bp, jp]],
                    sems, add=True)

            @pl.when(jnp.logical_and(j == CB - 1, blk + 1 < NB))
            def _():
                pltpu.make_async_copy(
                    src_hbm.at[wid, 0], sidx.at[0], semi).wait()
                pltpu.make_async_copy(
                    dst_hbm.at[wid, 0], didx.at[0], semi).wait()

            return 0
        lax.fori_loop(0, NCH + 1, chunk, 0)
        pltpu.make_async_copy(
            rows.at[0], acc.at[didx.at[0, 0]], sems).wait()
        plsc.subcore_barrier()

        @pl.when(c == 0)
        def _():
            for off in ZOFF:
                sl = pl.ds(base_r + off, K)
                pltpu.async_copy(acc.at[sl], agg0.at[sl], semg0)
            for off in ZOFF:
                pltpu.make_async_copy(
                    acc.at[pl.ds(base_r, K)],
                    agg0.at[pl.ds(base_r, K)], semg0).wait()

        @pl.when(c == 1)
        def _():
            for off in ZOFF:
                sl = pl.ds(base_r + off, K)
                pltpu.async_copy(acc.at[sl], agg1.at[sl], semg0)
            for off in ZOFF:
                pltpu.make_async_copy(
                    acc.at[pl.ds(base_r, K)],
                    agg1.at[pl.ds(base_r, K)], semg0).wait()

    return agg_kernel


def _make_prescale(pad_cnt):
    full, rem = pad_cnt // K, pad_cnt % K

    def _prescale_body(x_ref, a_ref, b_ref, o_ref):
        deg = a_ref[...] + b_ref[...]
        row = lax.broadcasted_iota(jnp.int32, deg.shape, 0)
        on_first = pl.program_id(0) == 0
        corr = (jnp.where(jnp.logical_and(row < K, on_first), full, 0)
                + jnp.where(jnp.logical_and(row < rem, on_first), 1, 0))
        norm = lax.rsqrt(jnp.maximum(deg - corr.astype(jnp.float32), 1.0))
        o_ref[...] = x_ref[...] * norm
    return _prescale_body


def _final_body(a0_ref, a1_ref, d0_ref, d1_ref, w_ref, o_ref):
    agg = a0_ref[...] + a1_ref[...]
    deg = d0_ref[...] + d1_ref[...]
    norm = lax.rsqrt(jnp.maximum(deg, 1.0))
    o_ref[...] = jnp.dot(agg * norm, w_ref[...],
                         preferred_element_type=jnp.float32)


def kernel(x, edge_index, W):
    N, D = x.shape
    E = edge_index.shape[1]
    N_PAD = ((N + 1 + 127) // 128) * 128
    N_PADH = ((N + 1 + 255) // 256) * 256
    EB = NW * CB * K
    E_PAD = ((E + EB - 1) // EB) * EB
    pad_cnt = E_PAD - E
    assert N % BR == 0
    grid_n = N // BR

    src = edge_index[0]
    dst = edge_index[1]
    if pad_cnt:
        pad_src = jnp.asarray(np.arange(pad_cnt, dtype=np.int32) % K)
        pad_dst = jnp.asarray(
            N + np.arange(pad_cnt, dtype=np.int32) % (N_PAD - N))
        src = jnp.concatenate([src, pad_src])
        dst = jnp.concatenate([dst, pad_dst])
    src = src.reshape(NW, E_PAD // (NW * CB * K), CB, K)
    dst = dst.reshape(NW, E_PAD // (NW * CB * K), CB, K)

    hs0, hd0, hs1, hd1 = _build_hist(E_PAD, N_PADH)(src, dst)

    h = pl.pallas_call(
        _make_prescale(pad_cnt),
        grid=(grid_n,),
        in_specs=[
            pl.BlockSpec((BR, D), lambda i: (i, 0)),
            pl.BlockSpec((BR, 1), lambda i: (i, 0)),
            pl.BlockSpec((BR, 1), lambda i: (i, 0)),
        ],
        out_specs=pl.BlockSpec((BR, D), lambda i: (i, 0)),
        out_shape=jax.ShapeDtypeStruct((N, D), jnp.float32),
    )(x, hs0.reshape(-1, 1), hs1.reshape(-1, 1))

    agg0, agg1 = _build_agg(E_PAD, N, N_PAD, D)(h, src, dst)

    out = pl.pallas_call(
        _final_body,
        grid=(grid_n,),
        in_specs=[
            pl.BlockSpec((BR, D), lambda i: (i, 0)),
            pl.BlockSpec((BR, D), lambda i: (i, 0)),
            pl.BlockSpec((BR, 1), lambda i: (i, 0)),
            pl.BlockSpec((BR, 1), lambda i: (i, 0)),
            pl.BlockSpec((D, D), lambda i: (0, 0)),
        ],
        out_specs=pl.BlockSpec((BR, D), lambda i: (i, 0)),
        out_shape=jax.ShapeDtypeStruct((N, D), jnp.float32),
    )(agg0, agg1, hd0.reshape(-1, 1), hd1.reshape(-1, 1), W)

    return out


# speedup vs baseline: 3.2544x; 1.0111x over previous
import functools

import numpy as np

import jax
import jax.numpy as jnp
from jax import lax
from jax.experimental import pallas as pl
from jax.experimental.pallas import tpu as pltpu
from jax.experimental.pallas import tpu_sc as plsc

NC = 2
NS = 16
NW = NC * NS
LANES = 16
K = 64
CB = 8
BR = 2000


def _mesh():
    return plsc.VectorSubcoreMesh(core_axis_name="c", subcore_axis_name="s")


@functools.lru_cache(maxsize=None)
def _build_hist(E_PAD, N_PAD):
    EPW = E_PAD // NW
    NCH = EPW // K
    NB = NCH // CB
    ZH = N_PAD // NS
    ZF = ((ZH + LANES - 1) // LANES) * LANES
    FIRE = 4
    f32 = jnp.float32
    sds = jax.ShapeDtypeStruct

    @functools.partial(
        pl.kernel,
        out_type=(sds((N_PAD,), f32),) * 4,
        mesh=_mesh(),
        scratch_types=[
            pltpu.VMEM((NB, CB, K), jnp.int32),
            pltpu.VMEM((NB, CB, K), jnp.int32),
            pltpu.VMEM((K,), f32),
            pltpu.VMEM((ZF,), f32),
            pltpu.VMEM_SHARED((N_PAD,), f32),
            pltpu.VMEM_SHARED((N_PAD,), f32),
            pltpu.SemaphoreType.DMA,
        ],
    )
    def hist_kernel(src_hbm, dst_hbm, hs0, hd0, hs1, hd1,
                    sidx, didx, ones_v, zb, hist_s, hist_d, sem):
        c = lax.axis_index("c")
        s = lax.axis_index("s")
        wid = s * NC + c

        def fill_z(i, _):
            zb[pl.ds(i * LANES, LANES)] = jnp.zeros((LANES,), f32)
            return 0
        lax.fori_loop(0, ZF // LANES, fill_z, 0)

        def fill_o(i, _):
            ones_v[pl.ds(i * LANES, LANES)] = jnp.ones((LANES,), f32)
            return 0
        lax.fori_loop(0, K // LANES, fill_o, 0)

        r0 = pl.multiple_of(s * ZH, 8)
        pltpu.sync_copy(zb.at[pl.ds(0, ZH)], hist_s.at[pl.ds(r0, ZH)])
        pltpu.sync_copy(zb.at[pl.ds(0, ZH)], hist_d.at[pl.ds(r0, ZH)])

        pltpu.sync_copy(src_hbm.at[wid], sidx)
        pltpu.sync_copy(dst_hbm.at[wid], didx)
        plsc.subcore_barrier()

        def fire_block(ob, _):
            for j in range(CB):
                pltpu.async_copy(
                    ones_v, hist_s.at[sidx.at[ob, j]], sem, add=True)
                pltpu.async_copy(
                    ones_v, hist_d.at[didx.at[ob, j]], sem, add=True)
                if j % FIRE == FIRE - 1:
                    for _k in range(2 * FIRE):
                        pltpu.make_async_copy(
                            ones_v, hist_s.at[sidx.at[0, 0]], sem).wait()
            return 0
        lax.fori_loop(0, NB, fire_block, 0)
        plsc.subcore_barrier()

        @pl.when(c == 0)
        def _():
            pltpu.sync_copy(hist_s.at[pl.ds(r0, ZH)], hs0.at[pl.ds(r0, ZH)])
            pltpu.sync_copy(hist_d.at[pl.ds(r0, ZH)], hd0.at[pl.ds(r0, ZH)])

        @pl.when(c == 1)
        def _():
            pltpu.sync_copy(hist_s.at[pl.ds(r0, ZH)], hs1.at[pl.ds(r0, ZH)])
            pltpu.sync_copy(hist_d.at[pl.ds(r0, ZH)], hd1.at[pl.ds(r0, ZH)])

    return hist_kernel


@functools.lru_cache(maxsize=None)
def _build_agg(E_PAD, N, N_PAD, D):
    EPW = E_PAD // NW
    NCH = EPW // K
    NB = NCH // CB
    RPT = N_PAD // NS
    ZOFF = list(range(0, RPT - K + 1, K))
    if ZOFF[-1] != RPT - K:
        ZOFF.append(RPT - K)
    f32 = jnp.float32
    sds = jax.ShapeDtypeStruct

    @functools.partial(
        pl.kernel,
        out_type=(sds((N_PAD, D), f32), sds((N_PAD, D), f32)),
        mesh=_mesh(),
        scratch_types=[
            pltpu.VMEM((2, CB, K), jnp.int32),
            pltpu.VMEM((2, CB, K), jnp.int32),
            pltpu.VMEM((3, K, D), f32),
            pltpu.VMEM_SHARED((N_PAD, D), f32),
            pltpu.SemaphoreType.DMA,
            pltpu.SemaphoreType.DMA,
            pltpu.SemaphoreType.DMA,
            pltpu.SemaphoreType.DMA,
            pltpu.SemaphoreType.DMA,
            pltpu.SemaphoreType.DMA,
            pltpu.SemaphoreType.DMA,
        ],
    )
    def agg_kernel(h_hbm, src_hbm, dst_hbm, agg0, agg1,
                   sidx, didx, rows, acc,
                   semg0, semg1, semg2, sems0, sems1, sems2, semi):
        c = lax.axis_index("c")
        s = lax.axis_index("s")
        wid = s * NC + c

        def fill_z(r, _):
            for jj in range(D // LANES):
                rows[0, r, pl.ds(jj * LANES, LANES)] = jnp.zeros(
                    (LANES,), f32)
            return 0
        lax.fori_loop(0, K, fill_z, 0)

        base_r = s * RPT
        for off in ZOFF:
            pltpu.async_copy(
                rows.at[0], acc.at[pl.ds(base_r + off, K)], semg0)
        for off in ZOFF:
            pltpu.make_async_copy(
                rows.at[0], acc.at[pl.ds(base_r, K)], semg0).wait()

        pltpu.sync_copy(src_hbm.at[wid, 0], sidx.at[0])
        pltpu.sync_copy(dst_hbm.at[wid, 0], didx.at[0])
        plsc.subcore_barrier()

        semg = (semg0, semg1, semg2)
        sems = (sems0, sems1, sems2)

        def chunk(cc, _):
            p = lax.rem(cc, 3)
            q = lax.rem(cc + 1, 3)
            blk = cc // CB
            j = lax.rem(cc, CB)
            bb = lax.rem(blk, 2)
            j2 = lax.rem(cc - 2, CB)
            b2 = lax.rem((cc - 2) // CB, 2)

            for pi in range(3):
                @pl.when(jnp.logical_and(cc >= 3, p == pi))
                def _():
                    pltpu.make_async_copy(
                        rows.at[0], acc.at[didx.at[0, 0]], sems[pi]).wait()

            @pl.when(jnp.logical_and(
                j == 2, jnp.logical_and(blk + 1 < NB, cc < NCH)))
            def _():
                pltpu.async_copy(
                    src_hbm.at[wid, blk + 1], sidx.at[1 - bb], semi)
                pltpu.async_copy(
                    dst_hbm.at[wid, blk + 1], didx.at[1 - bb], semi)

            for pi in range(3):
                @pl.when(jnp.logical_and(cc < NCH, p == pi))
                def _():
                    pltpu.async_copy(
                        h_hbm.at[sidx.at[bb, j]], rows.at[pi], semg[pi])

            for pi in range(3):
                @pl.when(jnp.logical_and(cc >= 2, q == pi))
                def _():
                    pltpu.make_async_copy(
                        h_hbm.at[sidx.at[0, 0]], rows.at[0],
                        semg[pi]).wait()
                    pltpu.async_copy(
                        rows.at[pi], acc.at[didx.at[b2, j2]],
                        sems[pi], add=True)

            @pl.when(jnp.logical_and(
                j == CB - 1, jnp.logical_and(blk + 1 < NB, cc < NCH)))
            def _():
                pltpu.make_async_copy(
                    src_hbm.at[wid, 0], sidx.at[0], semi).wait()
                pltpu.make_async_copy(
                    dst_hbm.at[wid, 0], didx.at[0], semi).wait()

            return 0
        lax.fori_loop(0, NCH + 2, chunk, 0)
        pltpu.make_async_copy(
            rows.at[0], acc.at[didx.at[0, 0]],
            sems[(NCH - 1) % 3]).wait()
        plsc.subcore_barrier()

        @pl.when(c == 0)
        def _():
            for off in ZOFF:
                sl = pl.ds(base_r + off, K)
                pltpu.async_copy(acc.at[sl], agg0.at[sl], semg0)
            for off in ZOFF:
                pltpu.make_async_copy(
                    acc.at[pl.ds(base_r, K)],
                    agg0.at[pl.ds(base_r, K)], semg0).wait()

        @pl.when(c == 1)
        def _():
            for off in ZOFF:
                sl = pl.ds(base_r + off, K)
                pltpu.async_copy(acc.at[sl], agg1.at[sl], semg0)
            for off in ZOFF:
                pltpu.make_async_copy(
                    acc.at[pl.ds(base_r, K)],
                    agg1.at[pl.ds(base_r, K)], semg0).wait()

    return agg_kernel


def _make_prescale(pad_cnt):
    full, rem = pad_cnt // K, pad_cnt % K

    def _prescale_body(x_ref, a_ref, b_ref, o_ref):
        deg = a_ref[...] + b_ref[...]
        row = lax.broadcasted_iota(jnp.int32, deg.shape, 0)
        on_first = pl.program_id(0) == 0
        corr = (jnp.where(jnp.logical_and(row < K, on_first), full, 0)
                + jnp.where(jnp.logical_and(row < rem, on_first), 1, 0))
        norm = lax.rsqrt(jnp.maximum(deg - corr.astype(jnp.float32), 1.0))
        o_ref[...] = x_ref[...] * norm
    return _prescale_body


def _final_body(a0_ref, a1_ref, d0_ref, d1_ref, w_ref, o_ref):
    agg = a0_ref[...] + a1_ref[...]
    deg = d0_ref[...] + d1_ref[...]
    norm = lax.rsqrt(jnp.maximum(deg, 1.0))
    o_ref[...] = jnp.dot(agg * norm, w_ref[...],
                         preferred_element_type=jnp.float32)


def kernel(x, edge_index, W):
    N, D = x.shape
    E = edge_index.shape[1]
    N_PAD = ((N + 1 + 127) // 128) * 128
    N_PADH = ((N + 1 + 255) // 256) * 256
    EB = NW * CB * K
    E_PAD = ((E + EB - 1) // EB) * EB
    pad_cnt = E_PAD - E
    assert N % BR == 0
    grid_n = N // BR

    src = edge_index[0]
    dst = edge_index[1]
    if pad_cnt:
        pad_src = jnp.asarray(np.arange(pad_cnt, dtype=np.int32) % K)
        pad_dst = jnp.asarray(
            N + np.arange(pad_cnt, dtype=np.int32) % (N_PAD - N))
        src = jnp.concatenate([src, pad_src])
        dst = jnp.concatenate([dst, pad_dst])
    src = src.reshape(NW, E_PAD // (NW * CB * K), CB, K)
    dst = dst.reshape(NW, E_PAD // (NW * CB * K), CB, K)

    hs0, hd0, hs1, hd1 = _build_hist(E_PAD, N_PADH)(src, dst)

    h = pl.pallas_call(
        _make_prescale(pad_cnt),
        grid=(grid_n,),
        in_specs=[
            pl.BlockSpec((BR, D), lambda i: (i, 0)),
            pl.BlockSpec((BR, 1), lambda i: (i, 0)),
            pl.BlockSpec((BR, 1), lambda i: (i, 0)),
        ],
        out_specs=pl.BlockSpec((BR, D), lambda i: (i, 0)),
        out_shape=jax.ShapeDtypeStruct((N, D), jnp.float32),
    )(x, hs0.reshape(-1, 1), hs1.reshape(-1, 1))

    agg0, agg1 = _build_agg(E_PAD, N, N_PAD, D)(h, src, dst)

    out = pl.pallas_call(
        _final_body,
        grid=(grid_n,),
        in_specs=[
            pl.BlockSpec((BR, D), lambda i: (i, 0)),
            pl.BlockSpec((BR, D), lambda i: (i, 0)),
            pl.BlockSpec((BR, 1), lambda i: (i, 0)),
            pl.BlockSpec((BR, 1), lambda i: (i, 0)),
            pl.BlockSpec((D, D), lambda i: (0, 0)),
        ],
        out_specs=pl.BlockSpec((BR, D), lambda i: (i, 0)),
        out_shape=jax.ShapeDtypeStruct((N, D), jnp.float32),
    )(agg0, agg1, hd0.reshape(-1, 1), hd1.reshape(-1, 1), W)

    return out
